# baseline (device time: 159265 ns/iter reference)
import jax
import jax.numpy as jnp
from jax import lax
from jax.experimental import pallas as pl
from jax.experimental.pallas import tpu as pltpu

N_DEV = 16
SQ = 256
SKV = 4096
D_MODEL = 1024
HQ_LOCAL = 8
DH = 128
HD_LOCAL = HQ_LOCAL * DH
SCALE = 0.08838834764831843

_CompilerParams = getattr(pltpu, "CompilerParams", None) or getattr(
    pltpu, "TPUCompilerParams"
)


def _body(x_ref, wq_ref, k_ref, v_ref, wo_ref, out_ref,
          ctx_ref, comm_ref, send_sems, recv_sems):
    my = lax.axis_index("i")
    left = lax.rem(my + N_DEV - 1, N_DEV)
    right = lax.rem(my + 1, N_DEV)

    barrier_sem = pltpu.get_barrier_semaphore()
    for nbr in (left, right):
        pl.semaphore_signal(barrier_sem, inc=1, device_id=(nbr,),
                            device_id_type=pl.DeviceIdType.MESH)
    pl.semaphore_wait(barrier_sem, 2)

    q = jnp.dot(x_ref[...], wq_ref[...],
                preferred_element_type=jnp.float32)
    q = q.astype(jnp.bfloat16)

    qb = lax.broadcasted_iota(jnp.int32, (SQ, SKV), 0) // 64
    kb = lax.broadcasted_iota(jnp.int32, (SQ, SKV), 1) // 64
    mask = (kb % 4) == (qb % 4)

    for h in range(HQ_LOCAL):
        q_h = q[:, h * DH:(h + 1) * DH]
        s = lax.dot_general(q_h, k_ref[h], (((1,), (1,)), ((), ())),
                            preferred_element_type=jnp.float32)
        s = jnp.where(mask, s * SCALE, -1e9)
        m = jnp.max(s, axis=-1, keepdims=True)
        e = jnp.exp(s - m)
        w = (e / jnp.sum(e, axis=-1, keepdims=True)).astype(jnp.bfloat16)
        ctx_h = jnp.dot(w, v_ref[h], preferred_element_type=jnp.float32)
        ctx_ref[:, h * DH:(h + 1) * DH] = ctx_h.astype(jnp.bfloat16)

    partial = jnp.dot(ctx_ref[...], wo_ref[...],
                      preferred_element_type=jnp.float32)

    out_ref[...] = partial
    comm_ref[0] = partial.astype(jnp.bfloat16)

    for h in range(N_DEV - 1):
        rdma = pltpu.make_async_remote_copy(
            src_ref=comm_ref.at[h],
            dst_ref=comm_ref.at[h + 1],
            send_sem=send_sems.at[h],
            recv_sem=recv_sems.at[h],
            device_id=(right,),
            device_id_type=pl.DeviceIdType.MESH,
        )
        rdma.start()
        rdma.wait()
        out_ref[...] += comm_ref[h + 1].astype(jnp.float32)


def kernel(x, Wq, K_ext, V_ext, Wo):
    i = lax.axis_index("i")
    wq_s = lax.dynamic_slice(Wq, (0, i * HD_LOCAL), (D_MODEL, HD_LOCAL))
    wo_s = lax.dynamic_slice(Wo, (i * HD_LOCAL, 0), (HD_LOCAL, D_MODEL))
    x2 = x[0].astype(jnp.bfloat16)
    k3 = jnp.transpose(K_ext[0], (1, 0, 2)).astype(jnp.bfloat16)
    v3 = jnp.transpose(V_ext[0], (1, 0, 2)).astype(jnp.bfloat16)

    out2 = pl.pallas_call(
        _body,
        out_shape=jax.ShapeDtypeStruct((SQ, D_MODEL), jnp.float32),
        in_specs=[pl.BlockSpec(memory_space=pltpu.VMEM)] * 5,
        out_specs=pl.BlockSpec(memory_space=pltpu.VMEM),
        scratch_shapes=[
            pltpu.VMEM((SQ, HD_LOCAL), jnp.bfloat16),
            pltpu.VMEM((N_DEV, SQ, D_MODEL), jnp.bfloat16),
            pltpu.SemaphoreType.DMA((N_DEV - 1,)),
            pltpu.SemaphoreType.DMA((N_DEV - 1,)),
        ],
        compiler_params=_CompilerParams(collective_id=0),
    )(x2, wq_s.astype(jnp.bfloat16), k3, v3, wo_s.astype(jnp.bfloat16))
    return out2[None]


# device time: 61919 ns/iter; 2.5722x vs baseline; 2.5722x over previous
import jax
import jax.numpy as jnp
from jax import lax
from jax.experimental import pallas as pl
from jax.experimental.pallas import tpu as pltpu

N_DEV = 16
SQ = 256
SKV = 4096
D_MODEL = 1024
HQ_LOCAL = 8
DH = 128
HD_LOCAL = HQ_LOCAL * DH
SCALE = 0.08838834764831843

_CompilerParams = getattr(pltpu, "CompilerParams", None) or getattr(
    pltpu, "TPUCompilerParams"
)


ROWS = SQ // N_DEV


def _body(x_ref, wq_ref, k_ref, v_ref, wo_ref, out_ref,
          ctx_ref, rs_buf, ag_src, ag_buf,
          rs_send_sems, rs_recv_sems, ag_send_sems, ag_recv_sems):
    my = lax.axis_index("i")

    barrier_sem = pltpu.get_barrier_semaphore()
    for p in range(N_DEV):
        pl.semaphore_signal(barrier_sem, inc=1, device_id=(p,),
                            device_id_type=pl.DeviceIdType.MESH)
    pl.semaphore_wait(barrier_sem, N_DEV)

    q = jnp.dot(x_ref[...], wq_ref[...],
                preferred_element_type=jnp.float32)
    q = q.astype(jnp.bfloat16)

    qb = lax.broadcasted_iota(jnp.int32, (SQ, SKV), 0) // 64
    kb = lax.broadcasted_iota(jnp.int32, (SQ, SKV), 1) // 64
    mask = (kb % 4) == (qb % 4)

    for h in range(HQ_LOCAL):
        q_h = q[:, h * DH:(h + 1) * DH]
        s = lax.dot_general(q_h, k_ref[h], (((1,), (1,)), ((), ())),
                            preferred_element_type=jnp.float32)
        s = jnp.where(mask, s * SCALE, -1e9)
        m = jnp.max(s, axis=-1, keepdims=True)
        e = jnp.exp(s - m)
        w = (e / jnp.sum(e, axis=-1, keepdims=True)).astype(jnp.bfloat16)
        ctx_h = jnp.dot(w, v_ref[h], preferred_element_type=jnp.float32)
        ctx_ref[:, h * DH:(h + 1) * DH] = ctx_h.astype(jnp.bfloat16)

    partial = jnp.dot(ctx_ref[...], wo_ref[...],
                      preferred_element_type=jnp.float32)
    partial_bf = partial.astype(jnp.bfloat16)

    for j in range(N_DEV):
        rs_buf[j] = partial_bf[j * ROWS:(j + 1) * ROWS, :]

    rs_sends = []
    for j in range(N_DEV):
        rdma = pltpu.make_async_remote_copy(
            src_ref=rs_buf.at[j],
            dst_ref=ag_buf.at[my],
            send_sem=rs_send_sems.at[j],
            recv_sem=rs_recv_sems.at[my],
            device_id=(j,),
            device_id_type=pl.DeviceIdType.MESH,
        )
        rdma.start()
        rs_sends.append(rdma)

    for s in range(N_DEV):
        pltpu.make_async_remote_copy(
            src_ref=ag_buf.at[s], dst_ref=ag_buf.at[s],
            send_sem=rs_send_sems.at[s], recv_sem=rs_recv_sems.at[s],
            device_id=(0,), device_id_type=pl.DeviceIdType.MESH,
        ).wait_recv()
    red = ag_buf[0].astype(jnp.float32)
    for s in range(1, N_DEV):
        red = red + ag_buf[s].astype(jnp.float32)
    ag_src[...] = red.astype(jnp.bfloat16)

    ag_sends = []
    for j in range(N_DEV):
        rdma = pltpu.make_async_remote_copy(
            src_ref=ag_src,
            dst_ref=rs_buf.at[my],
            send_sem=ag_send_sems.at[j],
            recv_sem=ag_recv_sems.at[my],
            device_id=(j,),
            device_id_type=pl.DeviceIdType.MESH,
        )
        rdma.start()
        ag_sends.append(rdma)

    for j in range(N_DEV):
        pltpu.make_async_remote_copy(
            src_ref=rs_buf.at[j], dst_ref=rs_buf.at[j],
            send_sem=ag_send_sems.at[j], recv_sem=ag_recv_sems.at[j],
            device_id=(0,), device_id_type=pl.DeviceIdType.MESH,
        ).wait_recv()
        out_ref[j * ROWS:(j + 1) * ROWS, :] = rs_buf[j].astype(jnp.float32)

    for rdma in rs_sends + ag_sends:
        rdma.wait_send()


def kernel(x, Wq, K_ext, V_ext, Wo):
    i = lax.axis_index("i")
    wq_s = lax.dynamic_slice(Wq, (0, i * HD_LOCAL), (D_MODEL, HD_LOCAL))
    wo_s = lax.dynamic_slice(Wo, (i * HD_LOCAL, 0), (HD_LOCAL, D_MODEL))
    x2 = x[0].astype(jnp.bfloat16)
    k3 = jnp.transpose(K_ext[0], (1, 0, 2)).astype(jnp.bfloat16)
    v3 = jnp.transpose(V_ext[0], (1, 0, 2)).astype(jnp.bfloat16)

    out2 = pl.pallas_call(
        _body,
        out_shape=jax.ShapeDtypeStruct((SQ, D_MODEL), jnp.float32),
        in_specs=[pl.BlockSpec(memory_space=pltpu.VMEM)] * 5,
        out_specs=pl.BlockSpec(memory_space=pltpu.VMEM),
        scratch_shapes=[
            pltpu.VMEM((SQ, HD_LOCAL), jnp.bfloat16),
            pltpu.VMEM((N_DEV, ROWS, D_MODEL), jnp.bfloat16),
            pltpu.VMEM((ROWS, D_MODEL), jnp.bfloat16),
            pltpu.VMEM((N_DEV, ROWS, D_MODEL), jnp.bfloat16),
            pltpu.SemaphoreType.DMA((N_DEV,)),
            pltpu.SemaphoreType.DMA((N_DEV,)),
            pltpu.SemaphoreType.DMA((N_DEV,)),
            pltpu.SemaphoreType.DMA((N_DEV,)),
        ],
        compiler_params=_CompilerParams(collective_id=0),
    )(x2, wq_s.astype(jnp.bfloat16), k3, v3, wo_s.astype(jnp.bfloat16))
    return out2[None]
